# baseline (speedup 1.0000x reference)
import functools
import math

import jax
import jax.numpy as jnp
from jax import lax
from jax.experimental import pallas as pl
from jax.experimental.pallas import tpu as pltpu
from jax.experimental.pallas import tpu_sc as plsc

VOCAB = 100000
D = 1024
SCALE = math.sqrt(D)

NC, NS, L = 2, 16, 16
NW = NC * NS
XROWS, XCOLS = 4, 4096
B = XROWS * XCOLS
B_PER_W = B // NW
W_PER_XROW = XCOLS // B_PER_W
CHUNK = 16
NBUF = 4
DIST = 3
NCHUNK = B_PER_W // CHUNK

_mesh = plsc.VectorSubcoreMesh(core_axis_name="c", subcore_axis_name="s")


@functools.partial(
    pl.kernel,
    out_type=jax.ShapeDtypeStruct((B, D), jnp.float32),
    mesh=_mesh,
    scratch_types=(
        pltpu.VMEM((B_PER_W,), jnp.int32),
        [pltpu.VMEM((CHUNK, D), jnp.float32) for _ in range(NBUF)],
        [pltpu.SemaphoreType.DMA for _ in range(NBUF)],
        [pltpu.SemaphoreType.DMA for _ in range(NBUF)],
    ),
)
def _emb_kernel(x_hbm, table_hbm, out_hbm, idx_v, bufs, gsems, ssems):
    wid = lax.axis_index("s") * NC + lax.axis_index("c")
    base = wid * B_PER_W

    xr = wid // W_PER_XROW
    xc = (wid % W_PER_XROW) * B_PER_W
    pltpu.sync_copy(x_hbm.at[xr, pl.ds(xc, B_PER_W)], idx_v)

    def gather_desc(c, b):
        return pltpu.make_async_copy(
            table_hbm.at[idx_v.at[pl.ds(c * CHUNK, CHUNK)]], bufs[b], gsems[b])

    def store_desc(c, b):
        return pltpu.make_async_copy(
            bufs[b], out_hbm.at[pl.ds(base + c * CHUNK, CHUNK)], ssems[b])

    for c in range(DIST):
        gather_desc(c, c % NBUF).start()

    @pl.loop(0, NCHUNK, step=NBUF)
    def _group(g):
        for b in range(NBUF):
            c = g + b
            gather_desc(c, b).wait()

            buf = bufs[b]

            @plsc.parallel_loop(0, CHUNK * (D // L), unroll=8)
            def _scale(k):
                r = k // (D // L)
                sl = pl.ds((k % (D // L)) * L, L)
                buf[r, sl] = buf[r, sl] * SCALE

            store_desc(c, b).start()

            b2 = (b + DIST) % NBUF

            @pl.when(c + DIST < NCHUNK)
            def _():
                @pl.when(c + DIST - NBUF >= 0)
                def _():
                    store_desc(c + DIST - NBUF, b2).wait()
                gather_desc(c + DIST, b2).start()

    for c in range(NCHUNK - NBUF, NCHUNK):
        store_desc(c, c % NBUF).wait()


def kernel(x, table):
    out = _emb_kernel(x, table)
    return out.reshape(XROWS, XCOLS, D)


# speedup vs baseline: 1.0086x; 1.0086x over previous
import functools
import math

import jax
import jax.numpy as jnp
from jax import lax
from jax.experimental import pallas as pl
from jax.experimental.pallas import tpu as pltpu
from jax.experimental.pallas import tpu_sc as plsc

VOCAB = 100000
D = 1024
SCALE = math.sqrt(D)

NC, NS, L = 2, 16, 16
NW = NC * NS
XROWS, XCOLS = 4, 4096
B = XROWS * XCOLS
B_PER_W = B // NW
W_PER_XROW = XCOLS // B_PER_W
CHUNK = 8
NBUF = 8
DIST = 7
NCHUNK = B_PER_W // CHUNK
NSLOT = 4

_mesh = plsc.VectorSubcoreMesh(core_axis_name="c", subcore_axis_name="s")


@functools.partial(
    pl.kernel,
    out_type=jax.ShapeDtypeStruct((B, D), jnp.float32),
    mesh=_mesh,
    scratch_types=(
        pltpu.VMEM((B_PER_W,), jnp.int32),
        [pltpu.VMEM((CHUNK, D), jnp.float32) for _ in range(NBUF)],
        pltpu.VMEM_SHARED((NS, NSLOT, CHUNK, D), jnp.float32),
        [pltpu.SemaphoreType.DMA for _ in range(NBUF)],
        [pltpu.SemaphoreType.DMA for _ in range(NSLOT)],
        [pltpu.SemaphoreType.DMA for _ in range(NSLOT)],
    ),
)
def _emb_kernel(x_hbm, table_hbm, out_hbm, idx_v, bufs, spm, gsems, t2ssems, s2hsems):
    wid = lax.axis_index("s") * NC + lax.axis_index("c")
    base = wid * B_PER_W

    xr = wid // W_PER_XROW
    xc = (wid % W_PER_XROW) * B_PER_W
    pltpu.sync_copy(x_hbm.at[xr, pl.ds(xc, B_PER_W)], idx_v)

    def gather_desc(c, b):
        return pltpu.make_async_copy(
            table_hbm.at[idx_v.at[pl.ds(c * CHUNK, CHUNK)]], bufs[b], gsems[b])

    sid = lax.axis_index("s")

    def t2s_desc(b):
        sl = b % NSLOT
        return pltpu.make_async_copy(bufs[b], spm.at[sid, sl], t2ssems[sl])

    def s2h_desc(c, b):
        sl = b % NSLOT
        return pltpu.make_async_copy(
            spm.at[sid, sl], out_hbm.at[pl.ds(base + c * CHUNK, CHUNK)],
            s2hsems[sl])

    for c in range(DIST):
        gather_desc(c, c % NBUF).start()

    @pl.loop(0, NCHUNK, step=NBUF)
    def _group(g):
        for b in range(NBUF):
            c = g + b
            gather_desc(c, b).wait()

            buf = bufs[b]

            @plsc.parallel_loop(0, CHUNK * (D // L), unroll=8)
            def _scale(k):
                r = k // (D // L)
                sl = pl.ds((k % (D // L)) * L, L)
                buf[r, sl] = buf[r, sl] * SCALE

            @pl.when(c - NSLOT >= 0)
            def _():
                s2h_desc(c - NSLOT, b - NSLOT).wait()
            t2s_desc(b).start()

            bprev = (b - 1) % NBUF

            @pl.when(c - 1 >= 0)
            def _():
                t2s_desc(bprev).wait()
                s2h_desc(c - 1, bprev).start()

            @pl.when(c + DIST < NCHUNK)
            def _():
                gather_desc(c + DIST, (b + DIST) % NBUF).start()

    t2s_desc((NCHUNK - 1) % NBUF).wait()
    s2h_desc(NCHUNK - 1, (NCHUNK - 1) % NBUF).start()
    for c in range(NCHUNK - NSLOT, NCHUNK):
        s2h_desc(c, c % NBUF).wait()


def kernel(x, table):
    out = _emb_kernel(x, table)
    return out.reshape(XROWS, XCOLS, D)


# speedup vs baseline: 1.0175x; 1.0088x over previous
import functools
import math

import jax
import jax.numpy as jnp
from jax import lax
from jax.experimental import pallas as pl
from jax.experimental.pallas import tpu as pltpu
from jax.experimental.pallas import tpu_sc as plsc

VOCAB = 100000
D = 1024
SCALE = math.sqrt(D)

NC, NS, L = 2, 16, 16
NW = NC * NS
XROWS, XCOLS = 4, 4096
B = XROWS * XCOLS
B_PER_W = B // NW
W_PER_XROW = XCOLS // B_PER_W
CHUNK = 8
NBUF = 8
DIST = 7
NCHUNK = B_PER_W // CHUNK

_mesh = plsc.VectorSubcoreMesh(core_axis_name="c", subcore_axis_name="s")


@functools.partial(
    pl.kernel,
    out_type=jax.ShapeDtypeStruct((B, D), jnp.float32),
    mesh=_mesh,
    scratch_types=(
        pltpu.VMEM((B_PER_W,), jnp.int32),
        [pltpu.VMEM((CHUNK, D), jnp.float32) for _ in range(NBUF)],
        [pltpu.SemaphoreType.DMA for _ in range(NBUF)],
        [pltpu.SemaphoreType.DMA for _ in range(NBUF)],
    ),
)
def _emb_kernel(x_hbm, table_hbm, out_hbm, idx_v, bufs, gsems, ssems):
    wid = lax.axis_index("s") * NC + lax.axis_index("c")
    base = wid * B_PER_W

    xr = wid // W_PER_XROW
    xc = (wid % W_PER_XROW) * B_PER_W
    pltpu.sync_copy(x_hbm.at[xr, pl.ds(xc, B_PER_W)], idx_v)

    def gather_desc(c, b):
        return pltpu.make_async_copy(
            table_hbm.at[idx_v.at[pl.ds(c * CHUNK, CHUNK)]], bufs[b], gsems[b])

    def store_desc(c, b):
        return pltpu.make_async_copy(
            bufs[b], out_hbm.at[pl.ds(base + c * CHUNK, CHUNK)], ssems[b])

    for c in range(DIST):
        gather_desc(c, c % NBUF).start()

    @pl.loop(0, NCHUNK, step=NBUF)
    def _group(g):
        for b in range(NBUF):
            c = g + b
            gather_desc(c, b).wait()

            buf = bufs[b]

            @plsc.parallel_loop(0, CHUNK * (D // L), unroll=16)
            def _scale(k):
                r = k // (D // L)
                sl = pl.ds((k % (D // L)) * L, L)
                buf[r, sl] = buf[r, sl] * SCALE

            store_desc(c, b).start()

            b2 = (b + DIST) % NBUF

            @pl.when(c + DIST < NCHUNK)
            def _():
                @pl.when(c + DIST - NBUF >= 0)
                def _():
                    store_desc(c + DIST - NBUF, b2).wait()
                gather_desc(c + DIST, b2).start()

    for c in range(NCHUNK - NBUF, NCHUNK):
        store_desc(c, c % NBUF).wait()


def kernel(x, table):
    out = _emb_kernel(x, table)
    return out.reshape(XROWS, XCOLS, D)


# speedup vs baseline: 1.0201x; 1.0026x over previous
import functools
import math

import jax
import jax.numpy as jnp
from jax import lax
from jax.experimental import pallas as pl
from jax.experimental.pallas import tpu as pltpu
from jax.experimental.pallas import tpu_sc as plsc

VOCAB = 100000
D = 1024
SCALE = math.sqrt(D)

NC, NS, L = 2, 16, 16
NW = NC * NS
XROWS, XCOLS = 4, 4096
B = XROWS * XCOLS
B_PER_W = B // NW
W_PER_XROW = XCOLS // B_PER_W
CHUNK = 8
NBUF = 8
DIST = 7
NCHUNK = B_PER_W // CHUNK

_mesh = plsc.VectorSubcoreMesh(core_axis_name="c", subcore_axis_name="s")


@functools.partial(
    pl.kernel,
    out_type=jax.ShapeDtypeStruct((B, D), jnp.float32),
    mesh=_mesh,
    scratch_types=(
        pltpu.VMEM((B_PER_W,), jnp.int32),
        [pltpu.VMEM((CHUNK, D), jnp.float32) for _ in range(NBUF)],
        [pltpu.SemaphoreType.DMA for _ in range(NBUF)],
        [pltpu.SemaphoreType.DMA for _ in range(NBUF)],
    ),
)
def _emb_kernel(x_hbm, table_hbm, out_hbm, idx_v, bufs, gsems, ssems):
    wid = lax.axis_index("s") * NC + lax.axis_index("c")
    base = wid * B_PER_W

    xr = wid // W_PER_XROW
    xc = (wid % W_PER_XROW) * B_PER_W
    pltpu.sync_copy(x_hbm.at[xr, pl.ds(xc, B_PER_W)], idx_v)

    def gather_desc(c, b):
        return pltpu.make_async_copy(
            table_hbm.at[idx_v.at[pl.ds(c * CHUNK, CHUNK)]], bufs[b], gsems[b])

    def store_desc(c, b):
        return pltpu.make_async_copy(
            bufs[b], out_hbm.at[pl.ds(base + c * CHUNK, CHUNK)], ssems[b])

    for c in range(DIST):
        gather_desc(c, c % NBUF).start()

    @pl.loop(0, NCHUNK, step=NBUF)
    def _group(g):
        for b in range(NBUF):
            c = g + b
            gather_desc(c, b).wait()

            buf = bufs[b]

            @plsc.parallel_loop(0, CHUNK * (D // L), unroll=8)
            def _scale(k):
                r = k // (D // L)
                sl = pl.ds((k % (D // L)) * L, L)
                buf[r, sl] = buf[r, sl] * SCALE

            store_desc(c, b).start()

            b2 = (b + DIST) % NBUF

            @pl.when(c + DIST < NCHUNK)
            def _():
                @pl.when(c + DIST - NBUF >= 0)
                def _():
                    store_desc(c + DIST - NBUF, b2).wait()
                gather_desc(c + DIST, b2).start()

    for c in range(NCHUNK - NBUF, NCHUNK):
        store_desc(c, c % NBUF).wait()


def kernel(x, table):
    out = _emb_kernel(x, table)
    return out.reshape(XROWS, XCOLS, D)


# speedup vs baseline: 1.0243x; 1.0041x over previous
import functools
import math

import jax
import jax.numpy as jnp
from jax import lax
from jax.experimental import pallas as pl
from jax.experimental.pallas import tpu as pltpu
from jax.experimental.pallas import tpu_sc as plsc

VOCAB = 100000
D = 1024
SCALE = math.sqrt(D)

NC, NS, L = 2, 16, 16
NW = NC * NS
XROWS, XCOLS = 4, 4096
B = XROWS * XCOLS
B_PER_W = B // NW
W_PER_XROW = XCOLS // B_PER_W
CHUNK = 8
NBUF = 8
DIST = 7
NCHUNK = B_PER_W // CHUNK

@functools.cache
def _build_emb_kernel():
    mesh = plsc.VectorSubcoreMesh(
        core_axis_name="c", subcore_axis_name="s",
        num_cores=NC, num_subcores=NS)
    return pl.kernel(
        _emb_body,
        out_type=jax.ShapeDtypeStruct((B, D), jnp.float32),
        mesh=mesh,
        scratch_types=(
            pltpu.VMEM((B_PER_W,), jnp.int32),
            [pltpu.VMEM((CHUNK, D), jnp.float32) for _ in range(NBUF)],
            [pltpu.SemaphoreType.DMA for _ in range(NBUF)],
            [pltpu.SemaphoreType.DMA for _ in range(NBUF)],
        ),
    )


def _emb_body(x_hbm, table_hbm, out_hbm, idx_v, bufs, gsems, ssems):
    wid = lax.axis_index("s") * NC + lax.axis_index("c")
    base = wid * B_PER_W

    xr = wid // W_PER_XROW
    xc = (wid % W_PER_XROW) * B_PER_W
    pltpu.sync_copy(x_hbm.at[xr, pl.ds(xc, B_PER_W)], idx_v)

    def gather_desc(c, b):
        return pltpu.make_async_copy(
            table_hbm.at[idx_v.at[pl.ds(c * CHUNK, CHUNK)]], bufs[b], gsems[b])

    def store_desc(c, b):
        return pltpu.make_async_copy(
            bufs[b], out_hbm.at[pl.ds(base + c * CHUNK, CHUNK)], ssems[b])

    for c in range(DIST):
        gather_desc(c, c % NBUF).start()

    @pl.loop(0, NCHUNK, step=NBUF)
    def _group(g):
        for b in range(NBUF):
            c = g + b
            gather_desc(c, b).wait()

            buf = bufs[b]

            @plsc.parallel_loop(0, CHUNK * (D // L), unroll=8)
            def _scale(k):
                r = k // (D // L)
                sl = pl.ds((k % (D // L)) * L, L)
                buf[r, sl] = buf[r, sl] * SCALE

            store_desc(c, b).start()

            b2 = (b + DIST) % NBUF

            @pl.when(c + DIST < NCHUNK)
            def _():
                @pl.when(c + DIST - NBUF >= 0)
                def _():
                    store_desc(c + DIST - NBUF, b2).wait()
                gather_desc(c + DIST, b2).start()

    for c in range(NCHUNK - NBUF, NCHUNK):
        store_desc(c, c % NBUF).wait()


def kernel(x, table):
    out = _build_emb_kernel()(x, table)
    return out.reshape(XROWS, XCOLS, D)
